# baseline (speedup 1.0000x reference)
import functools

import jax
import jax.numpy as jnp
import numpy as np
from jax import lax
from jax.experimental import pallas as pl
from jax.experimental.pallas import tpu as pltpu
from jax.experimental.pallas import tpu_sc as plsc

_NUM_SEGMENTS = 100000
_N = 6400000
_NC = 2
_NS = 16
_L = 16
_NW = _NC * _NS
_SEG_PER_W = _NUM_SEGMENTS // _NW
_OBUF = 3200
_CHUNK = 8192
_UNROLL = 4

_NEG_INF = np.float32(-np.inf)


def _sc_body(data_hbm, ids_hbm, starts_hbm, out_hbm, sbuf, dbuf0, dbuf1,
             ibuf0, ibuf1, obufp, sd0, sd1, si0, si1):
    c = lax.axis_index("c")
    s = lax.axis_index("s")
    w = c * _NS + s
    s0 = w * _SEG_PER_W
    dbufs = (dbuf0, dbuf1)
    ibufs = (ibuf0, ibuf1)
    sd = (sd0, sd1)
    si = (si0, si1)

    pltpu.sync_copy(starts_hbm.at[w], sbuf)
    rng = sbuf[...]
    lo = rng[0]
    hi = rng[1]
    lo_al = lo & jnp.int32(-16)
    nchunks = (hi - lo_al + jnp.int32(_CHUNK - 1)) // jnp.int32(_CHUNK)

    def issue(k, b):
        base = jnp.minimum(lo_al + k * jnp.int32(_CHUNK), jnp.int32(_N - _CHUNK))
        base = pl.multiple_of(base, 8)
        pltpu.make_async_copy(data_hbm.at[pl.ds(base, _CHUNK)], dbufs[b], sd[b]).start()
        pltpu.make_async_copy(ids_hbm.at[pl.ds(base, _CHUNK)], ibufs[b], si[b]).start()

    def wait(b):
        pltpu.make_async_copy(data_hbm.at[pl.ds(0, _CHUNK)], dbufs[b], sd[b]).wait()
        pltpu.make_async_copy(ids_hbm.at[pl.ds(0, _CHUNK)], ibufs[b], si[b]).wait()

    issue(jnp.int32(0), 0)
    issue(jnp.int32(1), 1)

    neg = jnp.full((_L,), _NEG_INF, jnp.float32)

    def init_body(i, _):
        for u in range(8):
            obufp[pl.ds((i * 8 + u) * _L, _L)] = neg
        return 0

    lax.fori_loop(0, _L * _OBUF // _L // 8, init_body, 0)

    lanes = lax.iota(jnp.int32, _L)
    priv = lanes * _OBUF

    def flush(cid, cmax, extra_mask=None):
        fidx = cid - s0
        okf = fidx.astype(jnp.uint32) < jnp.uint32(_SEG_PER_W)
        fmask = okf if extra_mask is None else (okf & extra_mask)
        slot = jnp.where(fmask, priv + fidx, 0)
        old = plsc.load_gather(obufp, [slot], mask=fmask)
        plsc.store_scatter(obufp, [slot], jnp.maximum(old, cmax), mask=fmask)

    def one_vec(d_ref, i_ref, j, cid, cmax):
        g = i_ref[pl.ds(j * _L, _L)]
        v = d_ref[pl.ds(j * _L, _L)]
        changed = g != cid
        flush(cid, cmax, extra_mask=changed)
        new_max = jnp.where(changed, v, jnp.maximum(cmax, v))
        return g, new_max

    def compute(b, carry):
        d_ref = dbufs[b]
        i_ref = ibufs[b]

        def grp_body(t, carry):
            acc = list(carry)
            for u in range(_UNROLL):
                cid, cmax = acc[2 * u], acc[2 * u + 1]
                acc[2 * u], acc[2 * u + 1] = one_vec(
                    d_ref, i_ref, t * _UNROLL + u, cid, cmax
                )
            return tuple(acc)

        return lax.fori_loop(0, _CHUNK // _L // _UNROLL, grp_body, carry)

    def pair_body(gidx, carry):
        for b in (0, 1):
            k = gidx * 2 + b
            wait(b)
            carry = compute(b, carry)
            issue(k + 2, b)
        return carry

    gmax = (nchunks + 1) // 2
    init_id = jnp.full((_L,), -1, jnp.int32)
    carry0 = (init_id, neg) * _UNROLL
    carry = lax.fori_loop(0, gmax, pair_body, carry0)

    for u in range(_UNROLL):
        flush(carry[2 * u], carry[2 * u + 1])

    wait(0)
    wait(1)

    def merge_body(i, _):
        acc = obufp[pl.ds(i * _L, _L)]
        for l in range(1, _L):
            acc = jnp.maximum(acc, obufp[pl.ds(l * _OBUF + i * _L, _L)])
        obufp[pl.ds(i * _L, _L)] = acc
        return 0

    lax.fori_loop(0, _OBUF // _L, merge_body, 0)
    pltpu.sync_copy(obufp.at[pl.ds(0, _OBUF)], out_hbm.at[w])


@jax.jit
def _sc_segmax(data, ids, starts):
    mesh = plsc.VectorSubcoreMesh(
        core_axis_name="c", subcore_axis_name="s", num_cores=_NC, num_subcores=_NS
    )
    return pl.kernel(
        _sc_body,
        out_type=jax.ShapeDtypeStruct((_NW, _OBUF), jnp.float32),
        mesh=mesh,
        compiler_params=pltpu.CompilerParams(needs_layout_passes=False),
        scratch_types=[
            pltpu.VMEM((_L,), jnp.int32),
            pltpu.VMEM((_CHUNK,), jnp.float32),
            pltpu.VMEM((_CHUNK,), jnp.float32),
            pltpu.VMEM((_CHUNK,), jnp.int32),
            pltpu.VMEM((_CHUNK,), jnp.int32),
            pltpu.VMEM((_L * _OBUF,), jnp.float32),
            pltpu.SemaphoreType.DMA,
            pltpu.SemaphoreType.DMA,
            pltpu.SemaphoreType.DMA,
            pltpu.SemaphoreType.DMA,
        ],
    )(data, ids, starts)


def kernel(data, segment_ids):
    ids = segment_ids.astype(jnp.int32)
    bounds = jnp.arange(0, _NUM_SEGMENTS + 1, _SEG_PER_W, dtype=jnp.int32)
    edges = jnp.searchsorted(ids, bounds, side="left").astype(jnp.int32)
    starts = jnp.zeros((_NW, _L), jnp.int32)
    starts = starts.at[:, 0].set(edges[:-1]).at[:, 1].set(edges[1:])
    out = _sc_segmax(data, ids, starts)
    return out[:, :_SEG_PER_W].reshape(_NUM_SEGMENTS)


# speedup vs baseline: 1.7812x; 1.7812x over previous
import functools

import jax
import jax.numpy as jnp
import numpy as np
from jax import lax
from jax.experimental import pallas as pl
from jax.experimental.pallas import tpu as pltpu
from jax.experimental.pallas import tpu_sc as plsc

_NUM_SEGMENTS = 100000
_N = 6400000
_NC = 2
_NS = 16
_L = 16
_NW = _NC * _NS
_SEG_PER_W = _NUM_SEGMENTS // _NW
_OBUF = 3200
_CHUNK = 4096
_U = 2

_NEG_INF = np.float32(-np.inf)


def _sc_body(data_hbm, ids_hbm, starts_hbm, out_hbm, sbuf, dbuf0, dbuf1,
             ibuf0, ibuf1, ob0, ob1, sd0, sd1, si0, si1):
    c = lax.axis_index("c")
    s = lax.axis_index("s")
    w = c * _NS + s
    s0 = w * _SEG_PER_W
    dbufs = (dbuf0, dbuf1)
    ibufs = (ibuf0, ibuf1)
    obufs = (ob0, ob1)
    sd = (sd0, sd1)
    si = (si0, si1)

    pltpu.sync_copy(starts_hbm.at[w], sbuf)
    rng = sbuf[...]
    lo = rng[0]
    hi = rng[1]
    lo_al = lo & jnp.int32(-16)
    nchunks = (hi - lo_al + jnp.int32(_CHUNK - 1)) // jnp.int32(_CHUNK)

    def issue(k, b):
        base = jnp.minimum(lo_al + k * jnp.int32(_CHUNK), jnp.int32(_N - _CHUNK))
        base = pl.multiple_of(base, 8)
        pltpu.make_async_copy(data_hbm.at[pl.ds(base, _CHUNK)], dbufs[b], sd[b]).start()
        pltpu.make_async_copy(ids_hbm.at[pl.ds(base, _CHUNK)], ibufs[b], si[b]).start()

    def wait(b):
        pltpu.make_async_copy(data_hbm.at[pl.ds(0, _CHUNK)], dbufs[b], sd[b]).wait()
        pltpu.make_async_copy(ids_hbm.at[pl.ds(0, _CHUNK)], ibufs[b], si[b]).wait()

    issue(jnp.int32(0), 0)
    issue(jnp.int32(1), 1)

    neg = jnp.full((_L,), _NEG_INF, jnp.float32)

    def init_body(i, _):
        for u in range(8):
            ob0[pl.ds((i * 8 + u) * _L, _L)] = neg
            ob1[pl.ds((i * 8 + u) * _L, _L)] = neg
        return 0

    lax.fori_loop(0, _L * _OBUF // _L // 8, init_body, 0)

    lanes = lax.iota(jnp.int32, _L)
    priv = lanes * _OBUF

    def flush(ob, cid, cmax, extra_mask=None):
        fidx = cid - s0
        okf = fidx.astype(jnp.uint32) < jnp.uint32(_SEG_PER_W)
        fmask = okf if extra_mask is None else (okf & extra_mask)
        slot = jnp.where(fmask, priv + fidx, 0)
        plsc.store_scatter(ob, [slot], cmax, mask=fmask)

    def one_vec(d_ref, i_ref, ob, j, t0, cid, cmax):
        g = i_ref[pl.ds(j * _L, _L)]
        v = d_ref[pl.ds(j * _L, _L)]
        valid = j >= t0
        changed = (g != cid) & valid
        flush(ob, cid, cmax, extra_mask=changed)
        new_max = jnp.where(valid, jnp.where(changed, v, jnp.maximum(cmax, v)), cmax)
        new_id = jnp.where(valid, g, cid)
        return new_id, new_max

    def compute(b, k, carry):
        d_ref = dbufs[b]
        i_ref = ibufs[b]
        u_k = lo_al + k * jnp.int32(_CHUNK)
        base = jnp.minimum(u_k, jnp.int32(_N - _CHUNK))
        t0 = (u_k - base) // jnp.int32(_L)

        def grp_body(t, carry):
            acc = list(carry)
            for u in range(_U):
                cid, cmax = acc[2 * u], acc[2 * u + 1]
                acc[2 * u], acc[2 * u + 1] = one_vec(
                    d_ref, i_ref, obufs[u], t * _U + u, t0, cid, cmax
                )
            return tuple(acc)

        return lax.fori_loop(0, _CHUNK // _L // _U, grp_body, carry)

    def pair_body(gidx, carry):
        for b in (0, 1):
            k = gidx * 2 + b
            wait(b)
            carry = compute(b, k, carry)
            issue(k + 2, b)
        return carry

    gmax = (nchunks + 1) // 2
    init_id = jnp.full((_L,), -1, jnp.int32)
    carry0 = (init_id, neg) * _U
    carry = lax.fori_loop(0, gmax, pair_body, carry0)

    for u in range(_U):
        flush(obufs[u], carry[2 * u], carry[2 * u + 1])

    wait(0)
    wait(1)

    def merge_body(i, _):
        acc = ob0[pl.ds(i * _L, _L)]
        acc = jnp.maximum(acc, ob1[pl.ds(i * _L, _L)])
        for l in range(1, _L):
            acc = jnp.maximum(acc, ob0[pl.ds(l * _OBUF + i * _L, _L)])
            acc = jnp.maximum(acc, ob1[pl.ds(l * _OBUF + i * _L, _L)])
        ob0[pl.ds(i * _L, _L)] = acc
        return 0

    lax.fori_loop(0, _OBUF // _L, merge_body, 0)
    pltpu.sync_copy(ob0.at[pl.ds(0, _OBUF)], out_hbm.at[w])


@jax.jit
def _sc_segmax(data, ids, starts):
    mesh = plsc.VectorSubcoreMesh(
        core_axis_name="c", subcore_axis_name="s", num_cores=_NC, num_subcores=_NS
    )
    return pl.kernel(
        _sc_body,
        out_type=jax.ShapeDtypeStruct((_NW, _OBUF), jnp.float32),
        mesh=mesh,
        compiler_params=pltpu.CompilerParams(needs_layout_passes=False),
        scratch_types=[
            pltpu.VMEM((_L,), jnp.int32),
            pltpu.VMEM((_CHUNK,), jnp.float32),
            pltpu.VMEM((_CHUNK,), jnp.float32),
            pltpu.VMEM((_CHUNK,), jnp.int32),
            pltpu.VMEM((_CHUNK,), jnp.int32),
            pltpu.VMEM((_L * _OBUF,), jnp.float32),
            pltpu.VMEM((_L * _OBUF,), jnp.float32),
            pltpu.SemaphoreType.DMA,
            pltpu.SemaphoreType.DMA,
            pltpu.SemaphoreType.DMA,
            pltpu.SemaphoreType.DMA,
        ],
    )(data, ids, starts)


def kernel(data, segment_ids):
    ids = segment_ids.astype(jnp.int32)
    bounds = jnp.arange(0, _NUM_SEGMENTS + 1, _SEG_PER_W, dtype=jnp.int32)
    edges = jnp.searchsorted(ids, bounds, side="left").astype(jnp.int32)
    starts = jnp.zeros((_NW, _L), jnp.int32)
    starts = starts.at[:, 0].set(edges[:-1]).at[:, 1].set(edges[1:])
    out = _sc_segmax(data, ids, starts)
    return out[:, :_SEG_PER_W].reshape(_NUM_SEGMENTS)


# speedup vs baseline: 2.4678x; 1.3855x over previous
import functools

import jax
import jax.numpy as jnp
import numpy as np
from jax import lax
from jax.experimental import pallas as pl
from jax.experimental.pallas import tpu as pltpu
from jax.experimental.pallas import tpu_sc as plsc

_NUM_SEGMENTS = 100000
_N = 6400000
_NC = 2
_NS = 16
_L = 16
_NW = _NC * _NS
_SEG_PER_W = _NUM_SEGMENTS // _NW
_OBUF = 3200
_CHUNK = 4096
_U = 2

_NEG_INF = np.float32(-np.inf)


def _sc_body(data_hbm, ids_hbm, starts_hbm, out_hbm, sbuf, dbuf0, dbuf1,
             ibuf0, ibuf1, ob0, ob1, sd0, sd1, si0, si1):
    c = lax.axis_index("c")
    s = lax.axis_index("s")
    w = c * _NS + s
    s0 = w * _SEG_PER_W
    dbufs = (dbuf0, dbuf1)
    ibufs = (ibuf0, ibuf1)
    obufs = (ob0, ob1)
    sd = (sd0, sd1)
    si = (si0, si1)

    pltpu.sync_copy(starts_hbm.at[w], sbuf)
    rng = sbuf[...]
    lo = rng[0]
    hi = rng[1]
    lo_al = lo & jnp.int32(-16)
    nchunks = (hi - lo_al + jnp.int32(_CHUNK - 1)) // jnp.int32(_CHUNK)

    def issue(k, b):
        base = jnp.minimum(lo_al + k * jnp.int32(_CHUNK), jnp.int32(_N - _CHUNK))
        base = pl.multiple_of(base, 8)
        pltpu.make_async_copy(
            data_hbm.at[pl.ds(base, _CHUNK)], dbufs[b].at[pl.ds(0, _CHUNK)], sd[b]
        ).start()
        pltpu.make_async_copy(
            ids_hbm.at[pl.ds(base, _CHUNK)], ibufs[b].at[pl.ds(0, _CHUNK)], si[b]
        ).start()

    def wait(b):
        pltpu.make_async_copy(
            data_hbm.at[pl.ds(0, _CHUNK)], dbufs[b].at[pl.ds(0, _CHUNK)], sd[b]
        ).wait()
        pltpu.make_async_copy(
            ids_hbm.at[pl.ds(0, _CHUNK)], ibufs[b].at[pl.ds(0, _CHUNK)], si[b]
        ).wait()

    issue(jnp.int32(0), 0)
    issue(jnp.int32(1), 1)

    neg = jnp.full((_L,), _NEG_INF, jnp.float32)

    def init_body(i, _):
        for u in range(8):
            ob0[pl.ds((i * 8 + u) * _L, _L)] = neg
            ob1[pl.ds((i * 8 + u) * _L, _L)] = neg
        return 0

    lax.fori_loop(0, _L * _OBUF // _L // 8, init_body, 0)

    lanes = lax.iota(jnp.int32, _L)
    priv = lanes * _OBUF

    def flush(ob, cid, cmax, extra_mask=None):
        fidx = cid - s0
        okf = fidx.astype(jnp.uint32) < jnp.uint32(_SEG_PER_W)
        fmask = okf if extra_mask is None else (okf & extra_mask)
        slot = jnp.where(fmask, priv + fidx, 0)
        plsc.store_scatter(ob, [slot], cmax, mask=fmask)

    def one_vec(g, v, ob, j, t0, cid, cmax):
        valid = j >= t0
        changed = (g != cid) & valid
        flush(ob, cid, cmax, extra_mask=changed)
        new_max = jnp.where(valid, jnp.where(changed, v, jnp.maximum(cmax, v)), cmax)
        new_id = jnp.where(valid, g, cid)
        return new_id, new_max

    def compute(b, k, carry):
        d_ref = dbufs[b]
        i_ref = ibufs[b]
        u_k = lo_al + k * jnp.int32(_CHUNK)
        base = jnp.minimum(u_k, jnp.int32(_N - _CHUNK))
        t0 = (u_k - base) // jnp.int32(_L)

        def loads(t):
            j = t * _U
            return tuple(
                r[pl.ds((j + u) * _L, _L)]
                for u in range(_U)
                for r in (i_ref, d_ref)
            )

        def grp_body(t, state):
            acc = list(state[: 2 * _U])
            cur = state[2 * _U:]
            nxt = loads(t + 1)
            for u in range(_U):
                cid, cmax = acc[2 * u], acc[2 * u + 1]
                acc[2 * u], acc[2 * u + 1] = one_vec(
                    cur[2 * u], cur[2 * u + 1], obufs[u], t * _U + u, t0, cid, cmax
                )
            return tuple(acc) + nxt

        state = tuple(carry) + loads(jnp.int32(0))
        state = lax.fori_loop(0, _CHUNK // _L // _U, grp_body, state)
        return state[: 2 * _U]

    def pair_body(gidx, carry):
        for b in (0, 1):
            k = gidx * 2 + b
            wait(b)
            carry = compute(b, k, carry)
            issue(k + 2, b)
        return carry

    gmax = (nchunks + 1) // 2
    init_id = jnp.full((_L,), -1, jnp.int32)
    carry0 = (init_id, neg) * _U
    carry = lax.fori_loop(0, gmax, pair_body, carry0)

    for u in range(_U):
        flush(obufs[u], carry[2 * u], carry[2 * u + 1])

    wait(0)
    wait(1)

    def merge_body(i, _):
        acc = ob0[pl.ds(i * _L, _L)]
        acc = jnp.maximum(acc, ob1[pl.ds(i * _L, _L)])
        for l in range(1, _L):
            acc = jnp.maximum(acc, ob0[pl.ds(l * _OBUF + i * _L, _L)])
            acc = jnp.maximum(acc, ob1[pl.ds(l * _OBUF + i * _L, _L)])
        ob0[pl.ds(i * _L, _L)] = acc
        return 0

    lax.fori_loop(0, _OBUF // _L, merge_body, 0)
    pltpu.sync_copy(ob0.at[pl.ds(0, _OBUF)], out_hbm.at[w])


@jax.jit
def _sc_segmax(data, ids, starts):
    mesh = plsc.VectorSubcoreMesh(
        core_axis_name="c", subcore_axis_name="s", num_cores=_NC, num_subcores=_NS
    )
    return pl.kernel(
        _sc_body,
        out_type=jax.ShapeDtypeStruct((_NW, _OBUF), jnp.float32),
        mesh=mesh,
        compiler_params=pltpu.CompilerParams(needs_layout_passes=False),
        scratch_types=[
            pltpu.VMEM((_L,), jnp.int32),
            pltpu.VMEM((_CHUNK + _U * _L,), jnp.float32),
            pltpu.VMEM((_CHUNK + _U * _L,), jnp.float32),
            pltpu.VMEM((_CHUNK + _U * _L,), jnp.int32),
            pltpu.VMEM((_CHUNK + _U * _L,), jnp.int32),
            pltpu.VMEM((_L * _OBUF,), jnp.float32),
            pltpu.VMEM((_L * _OBUF,), jnp.float32),
            pltpu.SemaphoreType.DMA,
            pltpu.SemaphoreType.DMA,
            pltpu.SemaphoreType.DMA,
            pltpu.SemaphoreType.DMA,
        ],
    )(data, ids, starts)


def kernel(data, segment_ids):
    ids = segment_ids.astype(jnp.int32)
    bounds = jnp.arange(0, _NUM_SEGMENTS + 1, _SEG_PER_W, dtype=jnp.int32)
    edges = jnp.searchsorted(ids, bounds, side="left").astype(jnp.int32)
    starts = jnp.zeros((_NW, _L), jnp.int32)
    starts = starts.at[:, 0].set(edges[:-1]).at[:, 1].set(edges[1:])
    out = _sc_segmax(data, ids, starts)
    return out[:, :_SEG_PER_W].reshape(_NUM_SEGMENTS)
